# Initial kernel scaffold; baseline (speedup 1.0000x reference)
import jax
import jax.numpy as jnp
from jax.experimental import pallas as pl


def kernel(audio_tokens, tables, W, b, gamma, beta):
    raise NotImplementedError("write your pallas kernel here")


# speedup vs baseline: 1.9238x; 1.9238x over previous
import math
import functools

import jax
import jax.numpy as jnp
from jax import lax
from jax.experimental import pallas as pl
from jax.experimental.pallas import tpu as pltpu
from jax.experimental.pallas import tpu_sc as plsc

B, S, K, V, D = 4, 4096, 4, 2048, 1024
N = B * S
NW = 32
TOK_W = N // NW
C = 16
ROWS = C * K
NCH = TOK_W // C
SQRT_D = math.sqrt(D)


def _proj_body(t_ref, w_ref, p_ref):
    p_ref[0] = lax.dot_general(
        t_ref[0], w_ref[...],
        (((1,), (1,)), ((), ())),
        preferred_element_type=jnp.float32,
    ) * SQRT_D


def _project(tables, W):
    return pl.pallas_call(
        _proj_body,
        grid=(K,),
        in_specs=[
            pl.BlockSpec((1, V, D), lambda k: (k, 0, 0)),
            pl.BlockSpec((D, D), lambda k: (0, k)),
        ],
        out_specs=pl.BlockSpec((1, V, D), lambda k: (k, 0, 0)),
        out_shape=jax.ShapeDtypeStruct((K, V, D), jnp.float32),
    )(tables, W).reshape(K * V, D)


def _gather_sum_body(p_hbm, idx_hbm, y_hbm, idx_v, g_v, o_v, sem):
    wid = lax.axis_index("s") * 2 + lax.axis_index("c")
    idx0 = wid * (TOK_W * K)
    row0 = wid * TOK_W
    pltpu.sync_copy(idx_hbm.at[pl.ds(idx0, TOK_W * K)], idx_v)

    def chunk(c, carry):
        pltpu.async_copy(
            p_hbm.at[idx_v.at[pl.ds(c * ROWS, ROWS)]], g_v, sem
        ).wait()

        def lane(j, carry2):
            for t in range(C):
                acc = g_v[K * t, pl.ds(j * 16, 16)]
                for k in range(1, K):
                    acc = acc + g_v[K * t + k, pl.ds(j * 16, 16)]
                o_v[t, pl.ds(j * 16, 16)] = acc
            return carry2

        lax.fori_loop(0, D // 16, lane, 0)
        pltpu.sync_copy(o_v, y_hbm.at[pl.ds(row0 + c * C, C)])
        return carry

    lax.fori_loop(0, NCH, chunk, 0)


def _gather_sum(P_flat, flat_idx):
    mesh = plsc.VectorSubcoreMesh(core_axis_name="c", subcore_axis_name="s")
    f = pl.kernel(
        _gather_sum_body,
        out_type=jax.ShapeDtypeStruct((N, D), jnp.float32),
        mesh=mesh,
        scratch_types=[
            pltpu.VMEM((TOK_W * K,), jnp.int32),
            pltpu.VMEM((ROWS, D), jnp.float32),
            pltpu.VMEM((C, D), jnp.float32),
            pltpu.SemaphoreType.DMA,
        ],
    )
    return f(P_flat, flat_idx)


_RB = 512
_NB = N // _RB


def _ln_body(y_ref, base_ref, g_ref, b_ref, o_ref):
    x = y_ref[...] + base_ref[...]
    mu = jnp.mean(x, axis=1, keepdims=True)
    xc = x - mu
    var = jnp.mean(xc * xc, axis=1, keepdims=True)
    o_ref[...] = xc * lax.rsqrt(var + 1e-5) * g_ref[...] + b_ref[...]


def _ln(y, base, gamma, beta):
    sb = S // _RB
    return pl.pallas_call(
        _ln_body,
        grid=(_NB,),
        in_specs=[
            pl.BlockSpec((_RB, D), lambda i: (i, 0)),
            pl.BlockSpec((_RB, D), lambda i: (i % sb, 0)),
            pl.BlockSpec((1, D), lambda i: (0, 0)),
            pl.BlockSpec((1, D), lambda i: (0, 0)),
        ],
        out_specs=pl.BlockSpec((_RB, D), lambda i: (i, 0)),
        out_shape=jax.ShapeDtypeStruct((N, D), jnp.float32),
    )(y, base, gamma.reshape(1, D), beta.reshape(1, D))


def _sin_pe():
    pos = jnp.arange(S, dtype=jnp.float32)[:, None]
    div = jnp.exp(
        jnp.arange(0, D, 2, dtype=jnp.float32) * (-math.log(10000.0) / D)
    )
    ang = pos * div
    return jnp.stack([jnp.sin(ang), jnp.cos(ang)], axis=-1).reshape(S, D)


def kernel(audio_tokens, tables, W, b, gamma, beta):
    tok = audio_tokens.astype(jnp.int32).reshape(N, K)
    flat_idx = (tok + (jnp.arange(K, dtype=jnp.int32) * V)[None, :]).reshape(-1)

    P = _project(tables, W)
    y = _gather_sum(P, flat_idx)
    base = _sin_pe() + b[None, :] * SQRT_D
    out = _ln(y, base, gamma, beta)
    return out.reshape(B, S, D)
